# baseline (speedup 1.0000x reference)
import functools

import jax
import jax.numpy as jnp
from jax import lax
from jax.experimental import pallas as pl
from jax.experimental.pallas import tpu as pltpu

def _pack4_kernel(m_ref, p_ref):
    grp = m_ref.shape[0] // 8
    for jj in range(2):
        acc = m_ref[pl.ds(jj * 4 * grp, grp), :].astype(jnp.int32)
        for g in range(1, 4):
            acc |= (m_ref[pl.ds((jj * 4 + g) * grp, grp), :]
                    .astype(jnp.int32) << g)
        p_ref[pl.ds(jj * grp, grp), :] = acc.astype(jnp.uint8)


def _layer0_kernel(x_ref, w_ref, mp_ref, b_ref, h_ref):
    k = pl.program_id(1)
    nk = pl.num_programs(1)

    @pl.when(k == 0)
    def _():
        h_ref[...] = jnp.zeros_like(h_ref)

    xb = x_ref[...].astype(jnp.bfloat16)
    mp = mp_ref[...].astype(jnp.int32)
    grp = w_ref.shape[0] // 4
    for g in range(4):
        sl = pl.ds(g * grp, grp)
        wg = jnp.where((mp << (31 - g)) < 0, w_ref[sl, :],
                       0.0).astype(jnp.bfloat16)
        pg = lax.dot_general(xb, wg, (((1,), (1,)), ((), ())),
                             preferred_element_type=jnp.float32)
        h_ref[:, sl] += pg

    @pl.when(k == nk - 1)
    def _():
        h_ref[...] = jnp.maximum(h_ref[...] + b_ref[...], 0.0)


def _tail_kernel(h_ref, w1_ref, m1_ref, b1_ref, w2_ref, m2_ref, b2_ref,
                 o_ref):
    w1b = jnp.where(m1_ref[...] != 0, w1_ref[...], 0.0).astype(jnp.bfloat16)
    h1 = lax.dot_general(h_ref[...].astype(jnp.bfloat16), w1b,
                         (((1,), (1,)), ((), ())),
                         preferred_element_type=jnp.float32)
    h1 = jnp.maximum(h1 + b1_ref[...], 0.0).astype(jnp.bfloat16)
    w2b = jnp.where(m2_ref[...] != 0, w2_ref[...], 0.0).astype(jnp.bfloat16)
    out = lax.dot_general(h1, w2b, (((1,), (1,)), ((), ())),
                          preferred_element_type=jnp.float32)
    o_ref[...] = out + b2_ref[...]


@functools.partial(jax.jit, static_argnames=("block_k",))
def _masked_mlp(x, W0, b0, W1, b1, W2, b2, mask0, mask1, mask2, block_k=1024):
    B, K0 = x.shape
    N0 = W0.shape[0]
    N1 = W1.shape[0]
    N2 = W2.shape[0]
    bk = min(block_k, K0)
    nk = K0 // bk

    m0 = mask0.view(jnp.uint8)
    m1 = mask1.view(jnp.uint8)
    m2 = mask2.view(jnp.uint8)

    grp = N0 // 8
    ck = min(4096, K0)
    mp0 = pl.pallas_call(
        _pack4_kernel,
        grid=(K0 // ck,),
        in_specs=[pl.BlockSpec((N0, ck), lambda c: (0, c))],
        out_specs=pl.BlockSpec((2 * grp, ck), lambda c: (0, c)),
        out_shape=jax.ShapeDtypeStruct((2 * grp, K0), jnp.uint8),
    )(m0)

    nj = 2
    bj = N0 // nj
    h0 = pl.pallas_call(
        _layer0_kernel,
        grid=(nj, nk),
        in_specs=[
            pl.BlockSpec((B, bk), lambda j, k: (0, k)),
            pl.BlockSpec((bj, bk), lambda j, k: (j, k)),
            pl.BlockSpec((grp, bk), lambda j, k: (j, k)),
            pl.BlockSpec((1, bj), lambda j, k: (0, j)),
        ],
        out_specs=pl.BlockSpec((B, bj), lambda j, k: (0, j)),
        out_shape=jax.ShapeDtypeStruct((B, N0), jnp.float32),
        compiler_params=pltpu.CompilerParams(
            dimension_semantics=("arbitrary", "arbitrary")),
    )(x, W0, mp0, b0.reshape(1, -1))

    full = lambda *s: pl.BlockSpec(s, lambda i: tuple(0 for _ in s))
    return pl.pallas_call(
        _tail_kernel,
        grid=(1,),
        in_specs=[
            full(B, N0),
            full(N1, N0), full(N1, N0), full(1, N1),
            full(N2, N1), full(N2, N1), full(1, N2),
        ],
        out_specs=full(B, N2),
        out_shape=jax.ShapeDtypeStruct((B, N2), jnp.float32),
    )(h0, W1, m1, b1.reshape(1, -1), W2, m2, b2.reshape(1, -1))


def kernel(x, W0, b0, W1, b1, W2, b2, mask0, mask1, mask2):
    return _masked_mlp(x, W0, b0, W1, b1, W2, b2, mask0, mask1, mask2)


# speedup vs baseline: 1.1330x; 1.1330x over previous
import functools

import jax
import jax.numpy as jnp
from jax import lax
from jax.experimental import pallas as pl
from jax.experimental.pallas import tpu as pltpu


def _layer0_kernel(x_ref, wa_ref, wb_ref, m_ref, b_ref, h_ref):
    k = pl.program_id(1)
    nk = pl.num_programs(1)

    @pl.when(k == 0)
    def _():
        h_ref[...] = jnp.zeros_like(h_ref)

    xb = x_ref[...].astype(jnp.bfloat16)
    half = wa_ref.shape[0]
    grp = half // 2
    for g in range(4):
        w_ref = wa_ref if g < 2 else wb_ref
        wsl = pl.ds((g % 2) * grp, grp)
        sl = pl.ds(g * grp, grp)
        wg = jnp.where(m_ref[sl, :] != 0, w_ref[wsl, :],
                       0.0).astype(jnp.bfloat16)
        pg = lax.dot_general(xb, wg, (((1,), (1,)), ((), ())),
                             preferred_element_type=jnp.float32)
        h_ref[:, sl] += pg

    @pl.when(k == nk - 1)
    def _():
        h_ref[...] = jnp.maximum(h_ref[...] + b_ref[...], 0.0)


def _tail_kernel(h_ref, w1_ref, m1_ref, b1_ref, w2_ref, m2_ref, b2_ref,
                 o_ref):
    w1b = jnp.where(m1_ref[...] != 0, w1_ref[...], 0.0).astype(jnp.bfloat16)
    h1 = lax.dot_general(h_ref[...].astype(jnp.bfloat16), w1b,
                         (((1,), (1,)), ((), ())),
                         preferred_element_type=jnp.float32)
    h1 = jnp.maximum(h1 + b1_ref[...], 0.0).astype(jnp.bfloat16)
    w2b = jnp.where(m2_ref[...] != 0, w2_ref[...], 0.0).astype(jnp.bfloat16)
    out = lax.dot_general(h1, w2b, (((1,), (1,)), ((), ())),
                          preferred_element_type=jnp.float32)
    o_ref[...] = out + b2_ref[...]


@functools.partial(jax.jit, static_argnames=("block_k",))
def _masked_mlp(x, W0, b0, W1, b1, W2, b2, mask0, mask1, mask2, block_k=1024):
    B, K0 = x.shape
    N0 = W0.shape[0]
    N1 = W1.shape[0]
    N2 = W2.shape[0]
    bk = min(block_k, K0)
    nk = K0 // bk

    m0 = mask0.view(jnp.uint8)
    m1 = mask1.view(jnp.uint8)
    m2 = mask2.view(jnp.uint8)

    nj = 2
    bj = N0 // nj
    h0 = pl.pallas_call(
        _layer0_kernel,
        grid=(nj, nk),
        in_specs=[
            pl.BlockSpec((B, bk), lambda j, k: (0, k)),
            pl.BlockSpec((bj // 2, bk), lambda j, k: (2 * j, k)),
            pl.BlockSpec((bj // 2, bk), lambda j, k: (2 * j + 1, k)),
            pl.BlockSpec((bj, bk), lambda j, k: (j, k)),
            pl.BlockSpec((1, bj), lambda j, k: (0, j)),
        ],
        out_specs=pl.BlockSpec((B, bj), lambda j, k: (0, j)),
        out_shape=jax.ShapeDtypeStruct((B, N0), jnp.float32),
        compiler_params=pltpu.CompilerParams(
            dimension_semantics=("arbitrary", "arbitrary")),
    )(x, W0, W0, m0, b0.reshape(1, -1))

    full = lambda *s: pl.BlockSpec(s, lambda i: tuple(0 for _ in s))
    return pl.pallas_call(
        _tail_kernel,
        grid=(1,),
        in_specs=[
            full(B, N0),
            full(N1, N0), full(N1, N0), full(1, N1),
            full(N2, N1), full(N2, N1), full(1, N2),
        ],
        out_specs=full(B, N2),
        out_shape=jax.ShapeDtypeStruct((B, N2), jnp.float32),
    )(h0, W1, m1, b1.reshape(1, -1), W2, m2, b2.reshape(1, -1))


def kernel(x, W0, b0, W1, b1, W2, b2, mask0, mask1, mask2):
    return _masked_mlp(x, W0, b0, W1, b1, W2, b2, mask0, mask1, mask2)


# speedup vs baseline: 1.1564x; 1.0207x over previous
import functools

import jax
import jax.numpy as jnp
from jax import lax
from jax.experimental import pallas as pl
from jax.experimental.pallas import tpu as pltpu


def _layer0_kernel(x_ref, wa_ref, wb_ref, m_ref, b_ref, h_ref, acc_ref):
    k = pl.program_id(1)
    nk = pl.num_programs(1)

    @pl.when(k == 0)
    def _():
        acc_ref[...] = jnp.zeros_like(acc_ref)

    xb = x_ref[...].astype(jnp.bfloat16)
    half = wa_ref.shape[0]
    grp = half // 2
    for g in range(4):
        w_ref = wa_ref if g < 2 else wb_ref
        wsl = pl.ds((g % 2) * grp, grp)
        sl = pl.ds(g * grp, grp)
        wg = jnp.where(m_ref[sl, :] != 0, w_ref[wsl, :],
                       0.0).astype(jnp.bfloat16)
        pg = lax.dot_general(xb, wg, (((1,), (1,)), ((), ())),
                             preferred_element_type=jnp.float32)
        acc_ref[:, sl] += pg

    @pl.when(k == nk - 1)
    def _():
        h_ref[...] = jnp.maximum(acc_ref[...] + b_ref[...],
                                 0.0).astype(jnp.bfloat16)


def _tail_kernel(h_ref, w1_ref, m1_ref, b1_ref, w2_ref, m2_ref, b2_ref,
                 o_ref):
    w1b = jnp.where(m1_ref[...] != 0, w1_ref[...], 0.0).astype(jnp.bfloat16)
    h1 = lax.dot_general(h_ref[...].astype(jnp.bfloat16), w1b,
                         (((1,), (1,)), ((), ())),
                         preferred_element_type=jnp.float32)
    h1 = jnp.maximum(h1 + b1_ref[...], 0.0).astype(jnp.bfloat16)
    w2b = jnp.where(m2_ref[...] != 0, w2_ref[...], 0.0).astype(jnp.bfloat16)
    out = lax.dot_general(h1, w2b, (((1,), (1,)), ((), ())),
                          preferred_element_type=jnp.float32)
    o_ref[...] = out + b2_ref[...]


@functools.partial(jax.jit, static_argnames=("block_k",))
def _masked_mlp(x, W0, b0, W1, b1, W2, b2, mask0, mask1, mask2, block_k=1024):
    B, K0 = x.shape
    N0 = W0.shape[0]
    N1 = W1.shape[0]
    N2 = W2.shape[0]
    bk = min(block_k, K0)
    nk = K0 // bk

    m0 = mask0.view(jnp.uint8)
    m1 = mask1.view(jnp.uint8)
    m2 = mask2.view(jnp.uint8)

    nj = 2
    bj = N0 // nj
    h0 = pl.pallas_call(
        _layer0_kernel,
        grid=(nj, nk),
        in_specs=[
            pl.BlockSpec((B, bk), lambda j, k: (0, k)),
            pl.BlockSpec((bj // 2, bk), lambda j, k: (2 * j, k)),
            pl.BlockSpec((bj // 2, bk), lambda j, k: (2 * j + 1, k)),
            pl.BlockSpec((bj, bk), lambda j, k: (j, k)),
            pl.BlockSpec((1, bj), lambda j, k: (0, j)),
        ],
        out_specs=pl.BlockSpec((B, bj), lambda j, k: (0, j)),
        out_shape=jax.ShapeDtypeStruct((B, N0), jnp.bfloat16),
        scratch_shapes=[pltpu.VMEM((B, bj), jnp.float32)],
        compiler_params=pltpu.CompilerParams(
            dimension_semantics=("arbitrary", "arbitrary")),
    )(x, W0, W0, m0, b0.reshape(1, -1))

    full = lambda *s: pl.BlockSpec(s, lambda i: tuple(0 for _ in s))
    return pl.pallas_call(
        _tail_kernel,
        grid=(1,),
        in_specs=[
            full(B, N0),
            full(N1, N0), full(N1, N0), full(1, N1),
            full(N2, N1), full(N2, N1), full(1, N2),
        ],
        out_specs=full(B, N2),
        out_shape=jax.ShapeDtypeStruct((B, N2), jnp.float32),
    )(h0, W1, m1, b1.reshape(1, -1), W2, m2, b2.reshape(1, -1))


def kernel(x, W0, b0, W1, b1, W2, b2, mask0, mask1, mask2):
    return _masked_mlp(x, W0, b0, W1, b1, W2, b2, mask0, mask1, mask2)


# speedup vs baseline: 1.1893x; 1.0284x over previous
import functools

import jax
import jax.numpy as jnp
from jax import lax
from jax.experimental import pallas as pl
from jax.experimental.pallas import tpu as pltpu


def _layer0_kernel(x_ref, wa_ref, wb_ref, m_ref, b_ref, h_ref, acc_ref):
    k = pl.program_id(1)
    nk = pl.num_programs(1)

    @pl.when(k == 0)
    def _():
        acc_ref[...] = jnp.zeros_like(acc_ref)

    xb = x_ref[...].astype(jnp.bfloat16)
    half = wa_ref.shape[0]
    ng = 2 * half // min(512, half)
    grp = 2 * half // ng
    for g in range(ng):
        w_ref = wa_ref if g < ng // 2 else wb_ref
        wsl = pl.ds((g % (ng // 2)) * grp, grp)
        sl = pl.ds(g * grp, grp)
        wg = jnp.where(m_ref[sl, :] != 0, w_ref[wsl, :],
                       0.0).astype(jnp.bfloat16)
        pg = lax.dot_general(xb, wg, (((1,), (1,)), ((), ())),
                             preferred_element_type=jnp.float32)
        acc_ref[:, sl] += pg

    @pl.when(k == nk - 1)
    def _():
        h_ref[...] = jnp.maximum(acc_ref[...] + b_ref[...],
                                 0.0).astype(jnp.bfloat16)


def _tail_kernel(h_ref, w1_ref, m1_ref, b1_ref, w2_ref, m2_ref, b2_ref,
                 o_ref):
    w1b = jnp.where(m1_ref[...] != 0, w1_ref[...], 0.0).astype(jnp.bfloat16)
    h1 = lax.dot_general(h_ref[...].astype(jnp.bfloat16), w1b,
                         (((1,), (1,)), ((), ())),
                         preferred_element_type=jnp.float32)
    h1 = jnp.maximum(h1 + b1_ref[...], 0.0).astype(jnp.bfloat16)
    w2b = jnp.where(m2_ref[...] != 0, w2_ref[...], 0.0).astype(jnp.bfloat16)
    out = lax.dot_general(h1, w2b, (((1,), (1,)), ((), ())),
                          preferred_element_type=jnp.float32)
    o_ref[...] = out + b2_ref[...]


@functools.partial(jax.jit, static_argnames=("block_k",))
def _masked_mlp(x, W0, b0, W1, b1, W2, b2, mask0, mask1, mask2, block_k=512):
    B, K0 = x.shape
    N0 = W0.shape[0]
    N1 = W1.shape[0]
    N2 = W2.shape[0]
    bk = min(block_k, K0)
    nk = K0 // bk

    m0 = mask0.view(jnp.uint8)
    m1 = mask1.view(jnp.uint8)
    m2 = mask2.view(jnp.uint8)

    nj = 1
    bj = N0 // nj
    h0 = pl.pallas_call(
        _layer0_kernel,
        grid=(nj, nk),
        in_specs=[
            pl.BlockSpec((B, bk), lambda j, k: (0, k)),
            pl.BlockSpec((bj // 2, bk), lambda j, k: (2 * j, k)),
            pl.BlockSpec((bj // 2, bk), lambda j, k: (2 * j + 1, k)),
            pl.BlockSpec((bj, bk), lambda j, k: (j, k)),
            pl.BlockSpec((1, bj), lambda j, k: (0, j)),
        ],
        out_specs=pl.BlockSpec((B, bj), lambda j, k: (0, j)),
        out_shape=jax.ShapeDtypeStruct((B, N0), jnp.bfloat16),
        scratch_shapes=[pltpu.VMEM((B, bj), jnp.float32)],
        compiler_params=pltpu.CompilerParams(
            dimension_semantics=("arbitrary", "arbitrary")),
    )(x, W0, W0, m0, b0.reshape(1, -1))

    full = lambda *s: pl.BlockSpec(s, lambda i: tuple(0 for _ in s))
    return pl.pallas_call(
        _tail_kernel,
        grid=(1,),
        in_specs=[
            full(B, N0),
            full(N1, N0), full(N1, N0), full(1, N1),
            full(N2, N1), full(N2, N1), full(1, N2),
        ],
        out_specs=full(B, N2),
        out_shape=jax.ShapeDtypeStruct((B, N2), jnp.float32),
    )(h0, W1, m1, b1.reshape(1, -1), W2, m2, b2.reshape(1, -1))


def kernel(x, W0, b0, W1, b1, W2, b2, mask0, mask1, mask2):
    return _masked_mlp(x, W0, b0, W1, b1, W2, b2, mask0, mask1, mask2)


# speedup vs baseline: 1.2010x; 1.0098x over previous
import functools

import jax
import jax.numpy as jnp
from jax import lax
from jax.experimental import pallas as pl
from jax.experimental.pallas import tpu as pltpu


def _layer0_kernel(x_ref, wa_ref, wb_ref, m_ref, b_ref, h_ref, acc_ref):
    k = pl.program_id(1)
    nk = pl.num_programs(1)

    @pl.when(k == 0)
    def _():
        acc_ref[...] = jnp.zeros_like(acc_ref)

    xb = x_ref[...].astype(jnp.bfloat16)
    half = wa_ref.shape[0]
    ng = 2 * half // min(512, half)
    grp = 2 * half // ng
    for g in range(ng):
        w_ref = wa_ref if g < ng // 2 else wb_ref
        wsl = pl.ds((g % (ng // 2)) * grp, grp)
        sl = pl.ds(g * grp, grp)
        wg = jnp.where(m_ref[sl, :] != 0, w_ref[wsl, :],
                       0.0).astype(jnp.bfloat16)
        pg = lax.dot_general(xb, wg, (((1,), (1,)), ((), ())),
                             preferred_element_type=jnp.float32)
        acc_ref[:, sl] += pg

    @pl.when(k == nk - 1)
    def _():
        h_ref[...] = jnp.maximum(acc_ref[...] + b_ref[...],
                                 0.0).astype(jnp.bfloat16)


def _tail_kernel(h_ref, w1_ref, m1_ref, b1_ref, w2_ref, m2_ref, b2_ref,
                 o_ref, acc_ref):
    k = pl.program_id(0)
    nk = pl.num_programs(0)
    w1b = jnp.where(m1_ref[...] != 0, w1_ref[...], 0.0).astype(jnp.bfloat16)
    part = lax.dot_general(h_ref[...], w1b, (((1,), (1,)), ((), ())),
                           preferred_element_type=jnp.float32)

    @pl.when(k == 0)
    def _():
        acc_ref[...] = part

    @pl.when(k > 0)
    def _():
        acc_ref[...] += part

    @pl.when(k == nk - 1)
    def _():
        h1 = jnp.maximum(acc_ref[...] + b1_ref[...],
                         0.0).astype(jnp.bfloat16)
        w2b = jnp.where(m2_ref[...] != 0, w2_ref[...],
                        0.0).astype(jnp.bfloat16)
        out = lax.dot_general(h1, w2b, (((1,), (1,)), ((), ())),
                              preferred_element_type=jnp.float32)
        o_ref[...] = out + b2_ref[...]


@functools.partial(jax.jit, static_argnames=("block_k",))
def _masked_mlp(x, W0, b0, W1, b1, W2, b2, mask0, mask1, mask2, block_k=512):
    B, K0 = x.shape
    N0 = W0.shape[0]
    N1 = W1.shape[0]
    N2 = W2.shape[0]
    bk = min(block_k, K0)
    nk = K0 // bk

    m0 = mask0.view(jnp.uint8)
    m1 = mask1.view(jnp.uint8)
    m2 = mask2.view(jnp.uint8)

    nj = 1
    bj = N0 // nj
    h0 = pl.pallas_call(
        _layer0_kernel,
        grid=(nj, nk),
        in_specs=[
            pl.BlockSpec((B, bk), lambda j, k: (0, k)),
            pl.BlockSpec((bj // 2, bk), lambda j, k: (2 * j, k)),
            pl.BlockSpec((bj // 2, bk), lambda j, k: (2 * j + 1, k)),
            pl.BlockSpec((bj, bk), lambda j, k: (j, k)),
            pl.BlockSpec((1, bj), lambda j, k: (0, j)),
        ],
        out_specs=pl.BlockSpec((B, bj), lambda j, k: (0, j)),
        out_shape=jax.ShapeDtypeStruct((B, N0), jnp.bfloat16),
        scratch_shapes=[pltpu.VMEM((B, bj), jnp.float32)],
        compiler_params=pltpu.CompilerParams(
            dimension_semantics=("arbitrary", "arbitrary")),
    )(x, W0, W0, m0, b0.reshape(1, -1))

    bt = min(1024, N0)
    nt = N0 // bt
    full = lambda *s: pl.BlockSpec(s, lambda i: tuple(0 for _ in s))
    return pl.pallas_call(
        _tail_kernel,
        grid=(nt,),
        in_specs=[
            pl.BlockSpec((B, bt), lambda k: (0, k)),
            pl.BlockSpec((N1, bt), lambda k: (0, k)),
            pl.BlockSpec((N1, bt), lambda k: (0, k)),
            full(1, N1),
            full(N2, N1), full(N2, N1), full(1, N2),
        ],
        out_specs=full(B, N2),
        out_shape=jax.ShapeDtypeStruct((B, N2), jnp.float32),
        scratch_shapes=[pltpu.VMEM((B, N1), jnp.float32)],
        compiler_params=pltpu.CompilerParams(
            dimension_semantics=("arbitrary",)),
    )(h0, W1, m1, b1.reshape(1, -1), W2, m2, b2.reshape(1, -1))


def kernel(x, W0, b0, W1, b1, W2, b2, mask0, mask1, mask2):
    return _masked_mlp(x, W0, b0, W1, b1, W2, b2, mask0, mask1, mask2)
